# baseline (speedup 1.0000x reference)
import functools

import jax
import jax.numpy as jnp
from jax import lax
from jax.experimental import pallas as pl
from jax.experimental.pallas import tpu as pltpu
from jax.experimental.pallas import tpu_sc as plsc

N = 10000
N_IN = 128
HID = 256
H = 2
C = 256
ROW_BLK = 1000

NPAD = 10112
E_IN = 320000
WIN = 112
NTILE = 16
NWIN = 180
EPR = NWIN * WIN
ETOT = EPR * NTILE
RPT = NPAD // NTILE
NSLOT = 3


def _i32(v):
    return jnp.asarray(v, dtype=jnp.int32)


def _pre_body(x_ref, w1_ref, b1_ref, g_ref, lb_ref, w2_ref, b2_ref,
              gatw_ref, asrc_ref, adst_ref, skw_ref, skb_ref,
              xpc_ref, cinit_ref, skip_ref, ab_ref, ws_ref):
    x = x_ref[...]
    h = jnp.dot(x, w1_ref[...], preferred_element_type=jnp.float32) + b1_ref[...]
    mu = jnp.mean(h, axis=-1, keepdims=True)
    var = jnp.mean((h - mu) ** 2, axis=-1, keepdims=True)
    h = (h - mu) * lax.rsqrt(var + 1e-5) * g_ref[...] + lb_ref[...]
    h = jnp.maximum(h, 0.0)
    h = jnp.dot(h, w2_ref[...], preferred_element_type=jnp.float32) + b2_ref[...]
    xp = jnp.dot(h, gatw_ref[...], preferred_element_type=jnp.float32)
    skip_ref[...] = jnp.dot(h, skw_ref[...], preferred_element_type=jnp.float32) + skb_ref[...]
    xph = xp.reshape(xp.shape[0], H, C)
    a_row = jnp.sum(xph * asrc_ref[...][None], axis=-1)
    b_row = jnp.sum(xph * adst_ref[...][None], axis=-1)
    aw = a_row + b_row
    ws_row = jnp.exp(jnp.where(aw > 0, aw, 0.2 * aw))
    dn = (((1,), (1,)), ((), ()))
    att2 = jnp.concatenate([asrc_ref[...], adst_ref[...]], axis=0)
    for hh in range(H):
        xp_h = xph[:, hh, :]
        a_col = lax.dot_general(att2[hh:hh + 1], xp_h, dn,
                                preferred_element_type=jnp.float32)
        b_col = lax.dot_general(att2[H + hh:H + hh + 1], xp_h, dn,
                                preferred_element_type=jnp.float32)
        ab_ref[0, hh, :] = a_col[0]
        ab_ref[0, H + hh, :] = b_col[0]
        awc = a_col + b_col
        ws_ref[0, hh, :] = jnp.exp(jnp.where(awc > 0, awc, 0.2 * awc))[0]
    for c in range(4):
        sl = xp[:, c * 128:(c + 1) * 128]
        xpc_ref[c] = sl
        cinit_ref[c] = sl * ws_row[:, c // 2][:, None]


def _pre(x, enc_w1, enc_b1, ln_g, ln_b, enc_w2, enc_b2, gat_w, att_src,
         att_dst, skip_w, skip_b):
    n = x.shape[0]
    grid = n // ROW_BLK
    full = lambda shape: pl.BlockSpec(shape, lambda i: tuple(_i32(0) for _ in shape))
    rows = lambda d: pl.BlockSpec((ROW_BLK, d), lambda i: (_i32(i), _i32(0)))
    chunks = pl.BlockSpec((4, ROW_BLK, 128), lambda i: (_i32(0), _i32(i), _i32(0)))
    heads = lambda d0: pl.BlockSpec((1, d0, ROW_BLK),
                                    lambda i: (_i32(i), _i32(0), _i32(0)))
    return pl.pallas_call(
        _pre_body,
        grid=(grid,),
        in_specs=[
            rows(N_IN),
            full((N_IN, HID)), full((HID,)), full((HID,)), full((HID,)),
            full((HID, HID)), full((HID,)),
            full((HID, H * C)), full((H, C)), full((H, C)),
            full((HID, H * C)), full((H * C,)),
        ],
        out_specs=[chunks, chunks, rows(H * C), heads(2 * H), heads(H)],
        out_shape=[
            jax.ShapeDtypeStruct((4, NPAD, 128), jnp.float32),
            jax.ShapeDtypeStruct((4, NPAD, 128), jnp.float32),
            jax.ShapeDtypeStruct((n, H * C), jnp.float32),
            jax.ShapeDtypeStruct((grid, 2 * H, ROW_BLK), jnp.float32),
            jax.ShapeDtypeStruct((grid, H, ROW_BLK), jnp.float32),
        ],
    )(x, enc_w1, enc_b1, ln_g, ln_b, enc_w2, enc_b2, gat_w, att_src,
      att_dst, skip_w, skip_b)


def _sc_edge(ei2, a2, b2, xpall, cinit, dinit):
    mesh = plsc.VectorSubcoreMesh(core_axis_name="c", subcore_axis_name="s")

    @functools.partial(
        pl.kernel,
        mesh=mesh,
        compiler_params=pltpu.CompilerParams(needs_layout_passes=False),
        out_type=[
            jax.ShapeDtypeStruct((4 * NPAD, 128), jnp.float32),
            jax.ShapeDtypeStruct((2 * NPAD,), jnp.float32),
        ],
        scratch_types=(
            [pltpu.VMEM((2 * WIN,), jnp.int32)] * NSLOT
            + [pltpu.VMEM((WIN,), jnp.int32)] * NSLOT
            + [pltpu.VMEM((WIN,), jnp.int32)] * NSLOT
            + [pltpu.VMEM((WIN,), jnp.float32)] * NSLOT
            + [pltpu.VMEM((WIN,), jnp.float32)] * NSLOT
            + [pltpu.VMEM((WIN,), jnp.float32)] * NSLOT
            + [pltpu.VMEM((WIN, 128), jnp.float32)] * NSLOT
            + [
                pltpu.VMEM((RPT,), jnp.float32),
                pltpu.VMEM_SHARED((NPAD, 128), jnp.float32),
                pltpu.VMEM_SHARED((NPAD,), jnp.float32),
                pltpu.VMEM_SHARED((NPAD,), jnp.float32),
                pltpu.VMEM_SHARED((NPAD,), jnp.float32),
            ]
            + [pltpu.SemaphoreType.DMA] * (5 * NSLOT)
        ),
    )
    def body(ei_hbm, a_hbm, b_hbm, xp_hbm, cinit_hbm, dinit_hbm,
             conv_hbm, den_hbm, *refs):
        sd_ws = refs[0:3]
        dsc_ws = refs[3:6]
        sadj_s = refs[6:9]
        aw_s = refs[9:12]
        bw_s = refs[12:15]
        w_s = refs[15:18]
        rows_s = refs[18:21]
        den_v, acc, den_s, a_s, b_s = refs[21:26]
        sem_i = refs[26:29]
        sem_ab = refs[29:32]
        sem_xp = refs[32:35]
        sem_sc = refs[35:38]
        sem_dn = refs[38:41]
        hid = lax.axis_index("c")
        sid = lax.axis_index("s")
        hoff = hid * NPAD
        ro = sid * RPT
        rows0 = rows_s[0]

        pltpu.sync_copy(a_hbm.at[pl.ds(hoff + ro, RPT)], den_v)
        pltpu.sync_copy(den_v, a_s.at[pl.ds(ro, RPT)])
        pltpu.sync_copy(b_hbm.at[pl.ds(hoff + ro, RPT)], den_v)
        pltpu.sync_copy(den_v, b_s.at[pl.ds(ro, RPT)])

        def start_idx(wi, slot):
            base = (sid * NWIN + wi) * (2 * WIN)
            pltpu.async_copy(ei_hbm.at[pl.ds(base, 2 * WIN)], sd_ws[slot],
                             sem_i[slot])

        def wait_idx(wi, slot):
            base = (sid * NWIN + wi) * (2 * WIN)
            pltpu.make_async_copy(ei_hbm.at[pl.ds(base, 2 * WIN)],
                                  sd_ws[slot], sem_i[slot]).wait()

        for p in range(2):
            cidx = hid * 2 + p
            coff = cidx * NPAD
            for q in range(0, RPT, WIN):
                sz = min(WIN, RPT - q)
                pltpu.sync_copy(cinit_hbm.at[pl.ds(coff + ro + q, sz)],
                                rows0.at[pl.ds(0, sz)])
                pltpu.sync_copy(rows0.at[pl.ds(0, sz)],
                                acc.at[pl.ds(ro + q, sz)])
            if p == 0:
                pltpu.sync_copy(dinit_hbm.at[pl.ds(hoff + ro, RPT)], den_v)
                pltpu.sync_copy(den_v, den_s.at[pl.ds(ro, RPT)])
            plsc.subcore_barrier()

            def start_gathers(slot):
                for k in range(WIN // 16):
                    sl = pl.ds(k * 16, 16)
                    sadj_s[slot][sl] = sd_ws[slot][sl] + coff
                    dsc_ws[slot][sl] = sd_ws[slot][pl.ds(WIN + k * 16, 16)]
                pltpu.async_copy(xp_hbm.at[sadj_s[slot]], rows_s[slot],
                                 sem_xp[slot])
                pltpu.async_copy(a_s.at[sd_ws[slot].at[pl.ds(0, WIN)]],
                                 aw_s[slot], sem_ab[slot])
                pltpu.async_copy(b_s.at[sd_ws[slot].at[pl.ds(WIN, WIN)]],
                                 bw_s[slot], sem_ab[slot])

            def wait_scatter(slot):
                pltpu.make_async_copy(rows_s[slot], acc.at[dsc_ws[slot]],
                                      sem_sc[slot]).wait()

            def wait_den(slot):
                pltpu.make_async_copy(w_s[slot], den_s.at[dsc_ws[slot]],
                                      sem_dn[slot]).wait()

            def process(slot):
                pltpu.make_async_copy(a_s.at[sd_ws[slot].at[pl.ds(0, WIN)]],
                                      aw_s[slot], sem_ab[slot]).wait()
                pltpu.make_async_copy(b_s.at[sd_ws[slot].at[pl.ds(WIN, WIN)]],
                                      bw_s[slot], sem_ab[slot]).wait()
                for k in range(WIN // 16):
                    sl = pl.ds(k * 16, 16)
                    al = aw_s[slot][sl] + bw_s[slot][sl]
                    w_s[slot][sl] = jnp.exp(jnp.where(al > 0, al, 0.2 * al))
                if p == 0:
                    pltpu.async_copy(w_s[slot], den_s.at[dsc_ws[slot]],
                                     sem_dn[slot], add=True)
                pltpu.make_async_copy(xp_hbm.at[sadj_s[slot]], rows_s[slot],
                                      sem_xp[slot]).wait()

                def scale_body(g, _):
                    base = g * 16
                    wvec = w_s[slot][pl.ds(base, 16)]
                    for l in range(16):
                        wj = wvec[l]
                        for k in range(8):
                            sl = pl.ds(k * 16, 16)
                            rows_s[slot][base + l, sl] = rows_s[slot][base + l, sl] * wj
                    return 0

                lax.fori_loop(_i32(0), _i32(WIN // 16), scale_body, 0,
                              unroll=False)
                pltpu.async_copy(rows_s[slot], acc.at[dsc_ws[slot]],
                                 sem_sc[slot], add=True)

            def win3(base_wi, first):
                for j in range(NSLOT):
                    wi = base_wi + j
                    slot = j
                    s2 = (j + 2) % NSLOT
                    nx2 = wi + 2

                    def pf():
                        wait_idx(nx2, s2)
                        if not (first and j == 0):
                            wait_scatter(s2)
                            if p == 0:
                                wait_den(s2)
                        start_gathers(s2)

                    if first:
                        pf()
                    else:
                        @pl.when(nx2 < NWIN)
                        def _():
                            pf()
                    process(slot)
                    nx3 = wi + 3
                    if first:
                        start_idx(nx3, slot)
                    else:
                        @pl.when(nx3 < NWIN)
                        def _():
                            start_idx(nx3, slot)

            for j in range(NSLOT):
                start_idx(_i32(j), j)
            for j in range(2):
                wait_idx(_i32(j), j)
                start_gathers(j)
            win3(_i32(0), True)

            def step(t, _):
                win3(t * 3, False)
                return 0

            lax.fori_loop(_i32(1), _i32(NWIN // 3), step, 0, unroll=False)
            for j in range(NSLOT):
                wait_scatter(j)
                if p == 0:
                    wait_den(j)
            plsc.subcore_barrier()
            for q in range(0, RPT, WIN):
                sz = min(WIN, RPT - q)
                pltpu.sync_copy(acc.at[pl.ds(ro + q, sz)],
                                rows0.at[pl.ds(0, sz)])
                pltpu.sync_copy(rows0.at[pl.ds(0, sz)],
                                conv_hbm.at[pl.ds(coff + ro + q, sz)])
            if p == 0:
                pltpu.sync_copy(den_s.at[pl.ds(ro, RPT)], den_v)
                pltpu.sync_copy(den_v, den_hbm.at[pl.ds(hoff + ro, RPT)])
            plsc.subcore_barrier()

    return body(ei2, a2, b2, xpall, cinit, dinit)


def _post_body(c0_ref, c1_ref, c2_ref, c3_ref, den_ref, skip_ref,
               gatb_ref, w1_ref, b1_ref, w2_ref, b2_ref, out_ref):
    den = den_ref[...]
    chunks = (c0_ref[...][0], c1_ref[...][0], c2_ref[...][0], c3_ref[...][0])
    heads = []
    for h in range(H):
        conv_h = jnp.concatenate([chunks[2 * h], chunks[2 * h + 1]], axis=-1)
        heads.append(conv_h / den[:, h:h + 1])
    out = jnp.concatenate(heads, axis=-1) + gatb_ref[...] + skip_ref[...]
    out = jnp.where(out > 0, out, 0.1 * (jnp.exp(out) - 1.0))
    d = jnp.dot(out.astype(jnp.bfloat16), w1_ref[...].astype(jnp.bfloat16),
                preferred_element_type=jnp.float32) + b1_ref[...]
    d = jnp.where(d > 0, d, 0.1 * d)
    out_ref[...] = jnp.dot(d.astype(jnp.bfloat16),
                           w2_ref[...].astype(jnp.bfloat16),
                           preferred_element_type=jnp.float32) + b2_ref[...]


def _post(conv4, den2, skip, gat_b, dec_w1, dec_b1, dec_w2, dec_b2):
    n = skip.shape[0]
    grid = n // ROW_BLK
    full = lambda shape: pl.BlockSpec(shape, lambda i: tuple(_i32(0) for _ in shape))
    rows = lambda d: pl.BlockSpec((ROW_BLK, d), lambda i: (_i32(i), _i32(0)))
    chunk = lambda c: pl.BlockSpec((1, ROW_BLK, 128),
                                   lambda i, c=c: (_i32(c), _i32(i), _i32(0)))
    return pl.pallas_call(
        _post_body,
        grid=(grid,),
        in_specs=[
            chunk(0), chunk(1), chunk(2), chunk(3),
            rows(H), rows(H * C),
            full((H * C,)),
            full((H * C, 4 * HID)), full((4 * HID,)),
            full((4 * HID, 1)), full((1,)),
        ],
        out_specs=rows(1),
        out_shape=jax.ShapeDtypeStruct((n, 1), jnp.float32),
    )(conv4, conv4, conv4, conv4, den2, skip, gat_b, dec_w1, dec_b1,
      dec_w2, dec_b2)


def kernel(x, edge_index, batch_size, enc_w1, enc_b1, ln_g, ln_b, enc_w2,
           enc_b2, skip_w, skip_b, gat_w, att_src, att_dst, gat_b, dec_w1,
           dec_b1, dec_w2, dec_b2):
    n = x.shape[0]
    xpc, cinit, skip, ab, ws = _pre(x, enc_w1, enc_b1, ln_g, ln_b, enc_w2,
                                    enc_b2, gat_w, att_src, att_dst,
                                    skip_w, skip_b)

    src = edge_index[0].astype(jnp.int32)
    dst = edge_index[1].astype(jnp.int32)
    npadrows = NPAD - n
    padidx = n + (jnp.arange(ETOT - E_IN, dtype=jnp.int32) % npadrows)
    src_p = jnp.concatenate([src, padidx]).reshape(NTILE, NWIN, 1, WIN)
    dst_p = jnp.concatenate([dst, padidx]).reshape(NTILE, NWIN, 1, WIN)
    ei2 = jnp.concatenate([src_p, dst_p], axis=2).reshape(-1)

    abt = jnp.pad(ab.transpose(1, 0, 2).reshape(2 * H, n),
                  ((0, 0), (0, NPAD - n)))
    wst = jnp.pad(ws.transpose(1, 0, 2).reshape(H, n),
                  ((0, 0), (0, NPAD - n)))
    conv4, den = _sc_edge(ei2, abt[:H].reshape(-1),
                          abt[H:].reshape(-1), xpc.reshape(4 * NPAD, 128),
                          cinit.reshape(4 * NPAD, 128), wst.reshape(-1))
    den2 = den.reshape(2, NPAD).T
    d = _post(conv4.reshape(4, NPAD, 128), den2, skip, gat_b, dec_w1,
              dec_b1, dec_w2, dec_b2)
    return lax.dynamic_slice_in_dim(d, batch_size - n, n, axis=0)


# speedup vs baseline: 1.0092x; 1.0092x over previous
import functools

import jax
import jax.numpy as jnp
from jax import lax
from jax.experimental import pallas as pl
from jax.experimental.pallas import tpu as pltpu
from jax.experimental.pallas import tpu_sc as plsc

N = 10000
N_IN = 128
HID = 256
H = 2
C = 256
ROW_BLK = 1000

NPAD = 10112
E_IN = 320000
WIN = 112
NTILE = 16
NWIN = 180
EPR = NWIN * WIN
ETOT = EPR * NTILE
RPT = NPAD // NTILE
NSLOT = 3


def _i32(v):
    return jnp.asarray(v, dtype=jnp.int32)


def _pre_body(x_ref, w1_ref, b1_ref, g_ref, lb_ref, w2_ref, b2_ref,
              gatw_ref, asrc_ref, adst_ref, skw_ref, skb_ref,
              xpc_ref, cinit_ref, skip_ref, ab_ref, ws_ref):
    x = x_ref[...]
    h = jnp.dot(x, w1_ref[...], preferred_element_type=jnp.float32) + b1_ref[...]
    mu = jnp.mean(h, axis=-1, keepdims=True)
    var = jnp.mean((h - mu) ** 2, axis=-1, keepdims=True)
    h = (h - mu) * lax.rsqrt(var + 1e-5) * g_ref[...] + lb_ref[...]
    h = jnp.maximum(h, 0.0)
    h = jnp.dot(h, w2_ref[...], preferred_element_type=jnp.float32) + b2_ref[...]
    xp = jnp.dot(h, gatw_ref[...], preferred_element_type=jnp.float32)
    skip_ref[...] = jnp.dot(h, skw_ref[...], preferred_element_type=jnp.float32) + skb_ref[...]
    xph = xp.reshape(xp.shape[0], H, C)
    a_row = jnp.sum(xph * asrc_ref[...][None], axis=-1)
    b_row = jnp.sum(xph * adst_ref[...][None], axis=-1)
    aw = a_row + b_row
    ws_row = jnp.exp(jnp.where(aw > 0, aw, 0.2 * aw))
    dn = (((1,), (1,)), ((), ()))
    att2 = jnp.concatenate([asrc_ref[...], adst_ref[...]], axis=0)
    for hh in range(H):
        xp_h = xph[:, hh, :]
        a_col = lax.dot_general(att2[hh:hh + 1], xp_h, dn,
                                preferred_element_type=jnp.float32)
        b_col = lax.dot_general(att2[H + hh:H + hh + 1], xp_h, dn,
                                preferred_element_type=jnp.float32)
        ab_ref[0, hh, :] = a_col[0]
        ab_ref[0, H + hh, :] = b_col[0]
        awc = a_col + b_col
        ws_ref[0, hh, :] = jnp.exp(jnp.where(awc > 0, awc, 0.2 * awc))[0]
    for c in range(4):
        sl = xp[:, c * 128:(c + 1) * 128]
        xpc_ref[c] = sl
        cinit_ref[c] = sl * ws_row[:, c // 2][:, None]


def _pre(x, enc_w1, enc_b1, ln_g, ln_b, enc_w2, enc_b2, gat_w, att_src,
         att_dst, skip_w, skip_b):
    n = x.shape[0]
    grid = n // ROW_BLK
    full = lambda shape: pl.BlockSpec(shape, lambda i: tuple(_i32(0) for _ in shape))
    rows = lambda d: pl.BlockSpec((ROW_BLK, d), lambda i: (_i32(i), _i32(0)))
    chunks = pl.BlockSpec((4, ROW_BLK, 128), lambda i: (_i32(0), _i32(i), _i32(0)))
    heads = lambda d0: pl.BlockSpec((1, d0, ROW_BLK),
                                    lambda i: (_i32(i), _i32(0), _i32(0)))
    return pl.pallas_call(
        _pre_body,
        grid=(grid,),
        in_specs=[
            rows(N_IN),
            full((N_IN, HID)), full((HID,)), full((HID,)), full((HID,)),
            full((HID, HID)), full((HID,)),
            full((HID, H * C)), full((H, C)), full((H, C)),
            full((HID, H * C)), full((H * C,)),
        ],
        out_specs=[chunks, chunks, rows(H * C), heads(2 * H), heads(H)],
        out_shape=[
            jax.ShapeDtypeStruct((4, NPAD, 128), jnp.float32),
            jax.ShapeDtypeStruct((4, NPAD, 128), jnp.float32),
            jax.ShapeDtypeStruct((n, H * C), jnp.float32),
            jax.ShapeDtypeStruct((grid, 2 * H, ROW_BLK), jnp.float32),
            jax.ShapeDtypeStruct((grid, H, ROW_BLK), jnp.float32),
        ],
    )(x, enc_w1, enc_b1, ln_g, ln_b, enc_w2, enc_b2, gat_w, att_src,
      att_dst, skip_w, skip_b)


def _sc_edge(src, dst, a2, b2, xpall, cinit, dinit):
    mesh = plsc.VectorSubcoreMesh(core_axis_name="c", subcore_axis_name="s")

    @functools.partial(
        pl.kernel,
        mesh=mesh,
        compiler_params=pltpu.CompilerParams(needs_layout_passes=False),
        out_type=[
            jax.ShapeDtypeStruct((4 * NPAD, 128), jnp.float32),
            jax.ShapeDtypeStruct((2 * NPAD,), jnp.float32),
        ],
        scratch_types=(
            [pltpu.VMEM((WIN,), jnp.int32)] * NSLOT
            + [pltpu.VMEM((WIN,), jnp.int32)] * NSLOT
            + [pltpu.VMEM((WIN,), jnp.int32)] * NSLOT
            + [pltpu.VMEM((WIN,), jnp.int32)] * NSLOT
            + [pltpu.VMEM((WIN,), jnp.float32)] * NSLOT
            + [pltpu.VMEM((WIN,), jnp.float32)] * NSLOT
            + [pltpu.VMEM((WIN,), jnp.float32)] * NSLOT
            + [pltpu.VMEM((WIN, 128), jnp.float32)] * NSLOT
            + [
                pltpu.VMEM((RPT,), jnp.float32),
                pltpu.VMEM_SHARED((NPAD, 128), jnp.float32),
                pltpu.VMEM_SHARED((NPAD,), jnp.float32),
                pltpu.VMEM_SHARED((NPAD,), jnp.float32),
                pltpu.VMEM_SHARED((NPAD,), jnp.float32),
            ]
            + [pltpu.SemaphoreType.DMA] * (4 * NSLOT)
        ),
    )
    def body(src_hbm, dst_hbm, a_hbm, b_hbm, xp_hbm, cinit_hbm, dinit_hbm,
             conv_hbm, den_hbm, *refs):
        src_ws = refs[0:3]
        dst_ws = refs[3:6]
        dsc_ws = refs[6:9]
        sadj_s = refs[9:12]
        aw_s = refs[12:15]
        bw_s = refs[15:18]
        w_s = refs[18:21]
        rows_s = refs[21:24]
        den_v, acc, den_s, a_s, b_s = refs[24:29]
        sem_i = refs[29:32]
        sem_ab = refs[32:35]
        sem_xp = refs[35:38]
        sem_sc = refs[38:41]
        hid = lax.axis_index("c")
        sid = lax.axis_index("s")
        hoff = hid * NPAD
        ro = sid * RPT
        rows0 = rows_s[0]

        pltpu.sync_copy(a_hbm.at[pl.ds(hoff + ro, RPT)], den_v)
        pltpu.sync_copy(den_v, a_s.at[pl.ds(ro, RPT)])
        pltpu.sync_copy(b_hbm.at[pl.ds(hoff + ro, RPT)], den_v)
        pltpu.sync_copy(den_v, b_s.at[pl.ds(ro, RPT)])

        def start_idx(wi, slot):
            base = sid * EPR + wi * WIN
            pltpu.async_copy(src_hbm.at[pl.ds(base, WIN)], src_ws[slot],
                             sem_i[slot])
            pltpu.async_copy(dst_hbm.at[pl.ds(base, WIN)], dst_ws[slot],
                             sem_i[slot])

        def wait_idx(wi, slot):
            base = sid * EPR + wi * WIN
            pltpu.make_async_copy(src_hbm.at[pl.ds(base, WIN)], src_ws[slot],
                                  sem_i[slot]).wait()
            pltpu.make_async_copy(dst_hbm.at[pl.ds(base, WIN)], dst_ws[slot],
                                  sem_i[slot]).wait()

        for p in range(2):
            cidx = hid * 2 + p
            coff = cidx * NPAD
            for q in range(0, RPT, WIN):
                sz = min(WIN, RPT - q)
                pltpu.sync_copy(cinit_hbm.at[pl.ds(coff + ro + q, sz)],
                                rows0.at[pl.ds(0, sz)])
                pltpu.sync_copy(rows0.at[pl.ds(0, sz)],
                                acc.at[pl.ds(ro + q, sz)])
            if p == 0:
                pltpu.sync_copy(dinit_hbm.at[pl.ds(hoff + ro, RPT)], den_v)
                pltpu.sync_copy(den_v, den_s.at[pl.ds(ro, RPT)])
            plsc.subcore_barrier()

            def start_gathers(slot):
                for k in range(WIN // 16):
                    sl = pl.ds(k * 16, 16)
                    sadj_s[slot][sl] = src_ws[slot][sl] + coff
                pltpu.async_copy(xp_hbm.at[sadj_s[slot]], rows_s[slot],
                                 sem_xp[slot])
                pltpu.async_copy(a_s.at[src_ws[slot]], aw_s[slot],
                                 sem_ab[slot])
                pltpu.async_copy(b_s.at[dst_ws[slot]], bw_s[slot],
                                 sem_ab[slot])

            def wait_scatter(slot):
                pltpu.make_async_copy(rows_s[slot], acc.at[dsc_ws[slot]],
                                      sem_sc[slot]).wait()

            def process(slot):
                pltpu.make_async_copy(a_s.at[src_ws[slot]], aw_s[slot],
                                      sem_ab[slot]).wait()
                pltpu.make_async_copy(b_s.at[dst_ws[slot]], bw_s[slot],
                                      sem_ab[slot]).wait()
                for k in range(WIN // 16):
                    sl = pl.ds(k * 16, 16)
                    al = aw_s[slot][sl] + bw_s[slot][sl]
                    w_s[slot][sl] = jnp.exp(jnp.where(al > 0, al, 0.2 * al))
                    dsc_ws[slot][sl] = dst_ws[slot][sl]
                if p == 0:
                    pltpu.sync_copy(w_s[slot], den_s.at[dst_ws[slot]],
                                    add=True)
                pltpu.make_async_copy(xp_hbm.at[sadj_s[slot]], rows_s[slot],
                                      sem_xp[slot]).wait()

                def scale_body(g, _):
                    base = g * 16
                    wvec = w_s[slot][pl.ds(base, 16)]
                    for l in range(16):
                        wj = wvec[l]
                        for k in range(8):
                            sl = pl.ds(k * 16, 16)
                            rows_s[slot][base + l, sl] = rows_s[slot][base + l, sl] * wj
                    return 0

                lax.fori_loop(_i32(0), _i32(WIN // 16), scale_body, 0,
                              unroll=False)
                pltpu.async_copy(rows_s[slot], acc.at[dsc_ws[slot]],
                                 sem_sc[slot], add=True)

            def win3(base_wi, first):
                for j in range(NSLOT):
                    wi = base_wi + j
                    slot = j
                    s2 = (j + 2) % NSLOT
                    nx2 = wi + 2

                    def pf():
                        wait_idx(nx2, s2)
                        if not (first and j == 0):
                            wait_scatter(s2)
                        start_gathers(s2)

                    if first:
                        pf()
                    else:
                        @pl.when(nx2 < NWIN)
                        def _():
                            pf()
                    process(slot)
                    nx3 = wi + 3
                    if first:
                        start_idx(nx3, slot)
                    else:
                        @pl.when(nx3 < NWIN)
                        def _():
                            start_idx(nx3, slot)

            for j in range(NSLOT):
                start_idx(_i32(j), j)
            for j in range(2):
                wait_idx(_i32(j), j)
                start_gathers(j)
            win3(_i32(0), True)

            def step(t, _):
                win3(t * 3, False)
                return 0

            lax.fori_loop(_i32(1), _i32(NWIN // 3), step, 0, unroll=False)
            for j in range(NSLOT):
                wait_scatter(j)
            plsc.subcore_barrier()
            for q in range(0, RPT, WIN):
                sz = min(WIN, RPT - q)
                pltpu.sync_copy(acc.at[pl.ds(ro + q, sz)],
                                rows0.at[pl.ds(0, sz)])
                pltpu.sync_copy(rows0.at[pl.ds(0, sz)],
                                conv_hbm.at[pl.ds(coff + ro + q, sz)])
            if p == 0:
                pltpu.sync_copy(den_s.at[pl.ds(ro, RPT)], den_v)
                pltpu.sync_copy(den_v, den_hbm.at[pl.ds(hoff + ro, RPT)])
            plsc.subcore_barrier()

    return body(src, dst, a2, b2, xpall, cinit, dinit)


def _post_body(c0_ref, c1_ref, c2_ref, c3_ref, den_ref, skip_ref,
               gatb_ref, w1_ref, b1_ref, w2_ref, b2_ref, out_ref):
    den = den_ref[...]
    chunks = (c0_ref[...][0], c1_ref[...][0], c2_ref[...][0], c3_ref[...][0])
    heads = []
    for h in range(H):
        conv_h = jnp.concatenate([chunks[2 * h], chunks[2 * h + 1]], axis=-1)
        heads.append(conv_h / den[:, h:h + 1])
    out = jnp.concatenate(heads, axis=-1) + gatb_ref[...] + skip_ref[...]
    out = jnp.where(out > 0, out, 0.1 * (jnp.exp(out) - 1.0))
    d = jnp.dot(out, w1_ref[...], preferred_element_type=jnp.float32) + b1_ref[...]
    d = jnp.where(d > 0, d, 0.1 * d)
    out_ref[...] = jnp.dot(d, w2_ref[...], preferred_element_type=jnp.float32) + b2_ref[...]


def _post(conv4, den2, skip, gat_b, dec_w1, dec_b1, dec_w2, dec_b2):
    n = skip.shape[0]
    grid = n // ROW_BLK
    full = lambda shape: pl.BlockSpec(shape, lambda i: tuple(_i32(0) for _ in shape))
    rows = lambda d: pl.BlockSpec((ROW_BLK, d), lambda i: (_i32(i), _i32(0)))
    chunk = lambda c: pl.BlockSpec((1, ROW_BLK, 128),
                                   lambda i, c=c: (_i32(c), _i32(i), _i32(0)))
    return pl.pallas_call(
        _post_body,
        grid=(grid,),
        in_specs=[
            chunk(0), chunk(1), chunk(2), chunk(3),
            rows(H), rows(H * C),
            full((H * C,)),
            full((H * C, 4 * HID)), full((4 * HID,)),
            full((4 * HID, 1)), full((1,)),
        ],
        out_specs=rows(1),
        out_shape=jax.ShapeDtypeStruct((n, 1), jnp.float32),
    )(conv4, conv4, conv4, conv4, den2, skip, gat_b, dec_w1, dec_b1,
      dec_w2, dec_b2)


def kernel(x, edge_index, batch_size, enc_w1, enc_b1, ln_g, ln_b, enc_w2,
           enc_b2, skip_w, skip_b, gat_w, att_src, att_dst, gat_b, dec_w1,
           dec_b1, dec_w2, dec_b2):
    n = x.shape[0]
    xpc, cinit, skip, ab, ws = _pre(x, enc_w1, enc_b1, ln_g, ln_b, enc_w2,
                                    enc_b2, gat_w, att_src, att_dst,
                                    skip_w, skip_b)

    src = edge_index[0].astype(jnp.int32)
    dst = edge_index[1].astype(jnp.int32)
    npadrows = NPAD - n
    padidx = n + (jnp.arange(ETOT - E_IN, dtype=jnp.int32) % npadrows)
    src_p = jnp.concatenate([src, padidx])
    dst_p = jnp.concatenate([dst, padidx])

    abt = jnp.pad(ab.transpose(1, 0, 2).reshape(2 * H, n),
                  ((0, 0), (0, NPAD - n)))
    wst = jnp.pad(ws.transpose(1, 0, 2).reshape(H, n),
                  ((0, 0), (0, NPAD - n)))
    conv4, den = _sc_edge(src_p, dst_p, abt[:H].reshape(-1),
                          abt[H:].reshape(-1), xpc.reshape(4 * NPAD, 128),
                          cinit.reshape(4 * NPAD, 128), wst.reshape(-1))
    den2 = den.reshape(2, NPAD).T
    d = _post(conv4.reshape(4, NPAD, 128), den2, skip, gat_b, dec_w1,
              dec_b1, dec_w2, dec_b2)
    return lax.dynamic_slice_in_dim(d, batch_size - n, n, axis=0)
